# baseline (speedup 1.0000x reference)
import jax
import jax.numpy as jnp
from jax.experimental import pallas as pl
from jax.experimental.pallas import tpu as pltpu

NODE = 128
NMASK = 8
GRP = NODE // NMASK
DIM = 128
TOPK = 20
BSZ = 64
HIGH = jax.lax.Precision.HIGHEST


def _b16(v):
    return v.astype(jnp.bfloat16).astype(jnp.float32)


def _graph_kernel(emb_ref, s_ref):
    w = emb_ref[...]
    g = jax.lax.dot_general(w, w, (((1,), (1,)), ((), ())))
    nrm = jnp.sqrt(jnp.sum(w * w, axis=1))
    cos = g / (nrm[:, None] * nrm[None, :])
    colid = jax.lax.broadcasted_iota(jnp.int32, (NODE, NODE), 1)
    sel = jnp.zeros((NODE, NODE), jnp.float32)
    work = cos
    for _ in range(TOPK):
        rowmax = jnp.max(work, axis=1)
        ismax = work == rowmax[:, None]
        jidx = jnp.min(jnp.where(ismax, colid, NODE), axis=1)
        pick = colid == jidx[:, None]
        sel = jnp.where(pick, 1.0, sel)
        work = jnp.where(pick, -jnp.inf, work)
    s_ref[...] = sel


def _main_kernel(data_ref, ls_ref, s_ref, emb_ref, linw_ref, ai_ref, aj_ref,
                 aei_ref, aej_ref, w127_ref, mask8_ref, bias_ref,
                 agg_ref, s1_ref, q1_ref):
    b = pl.program_id(0)
    x = data_ref[0]
    base = jnp.dot(x, linw_ref[...], preferred_element_type=jnp.float32)
    ai = _b16(ai_ref[...])
    aj = _b16(aj_ref[...])
    wb = _b16(emb_ref[...])
    ei = jnp.sum(wb * _b16(aei_ref[...]), axis=1)
    ej = jnp.sum(wb * _b16(aej_ref[...]), axis=1)
    w127 = w127_ref[...]
    ls = ls_ref[0, 0]
    sel = s_ref[...]
    mask8 = mask8_ref[...]
    bias = bias_ref[...]
    ssum = jnp.zeros((NODE,), jnp.float32)
    sq = jnp.zeros((NODE,), jnp.float32)
    for m in range(NMASK):
        c = ls * mask8[m]
        xl = base + c[:, None] * w127
        xb = _b16(xl)
        si = jnp.sum(xb * ai, axis=1) + ei
        sj = jnp.sum(xb * aj, axis=1) + ej
        a = si[:, None] + sj[None, :]
        a = jnp.where(a >= 0, a, 0.2 * a)
        a = jnp.where(sel > 0, a, -1e30)
        amax = jnp.max(a, axis=1)
        ex = jnp.exp(a - amax[:, None])
        den = jnp.sum(ex, axis=1)
        rcp = 1.0 / (den + 1e-16)
        agg = jnp.dot(ex, xl, preferred_element_type=jnp.float32,
                      precision=HIGH) * rcp[:, None] + bias
        agg_ref[0, m] = agg.astype(jnp.bfloat16)
        ssum = ssum + jnp.sum(agg, axis=0)
        sq = sq + jnp.sum(agg * agg, axis=0)

    @pl.when(b == 0)
    def _init():
        s1_ref[...] = jnp.zeros_like(s1_ref)
        q1_ref[...] = jnp.zeros_like(q1_ref)

    s1_ref[...] += ssum[None, :]
    q1_ref[...] += sq[None, :]


def _bn_kernel(agg_ref, s1_ref, q1_ref, emb_ref, g1_ref, b1_ref, g2_ref,
               b2_ref, ow_ref, ob_ref, out_ref, usel_sc, s2_sc, q2_sc):
    b = pl.program_id(0)
    n = float(BSZ * NMASK * NODE)
    mean = s1_ref[...] / n
    var = q1_ref[...] / n - mean * mean
    inv = g1_ref[...] / jnp.sqrt(var + 1e-5)
    sh = b1_ref[...] - mean * inv
    emb = emb_ref[...]
    ssum = jnp.zeros((1, DIM), jnp.float32)
    sq = jnp.zeros((1, DIM), jnp.float32)
    for m in range(NMASK):
        h = jnp.maximum(agg_ref[0, m].astype(jnp.float32) * inv + sh, 0.0)
        u = h * emb
        ssum = ssum + jnp.sum(u, axis=0)[None, :]
        sq = sq + jnp.sum(u * u, axis=0)[None, :]
        usel_sc[b, m * GRP:(m + 1) * GRP, :] = u[m * GRP:(m + 1) * GRP, :]

    @pl.when(b == 0)
    def _init():
        s2_sc[...] = jnp.zeros_like(s2_sc)
        q2_sc[...] = jnp.zeros_like(q2_sc)

    s2_sc[...] += ssum
    q2_sc[...] += sq

    @pl.when(b == BSZ - 1)
    def _final():
        mean2 = s2_sc[...] / n
        var2 = q2_sc[...] / n - mean2 * mean2
        inv2 = g2_ref[...] / jnp.sqrt(var2 + 1e-5)
        sh2 = b2_ref[...] - mean2 * inv2
        y = jnp.maximum(usel_sc[...] * inv2[None] + sh2[None], 0.0)
        yb = _b16(y)
        wb = _b16(ow_ref[...])
        out_ref[...] = jnp.sum(yb * wb[None], axis=2) + ob_ref[0, 0]


def kernel(data, org_edge_index, last_state, embedding, lin_W, att_i, att_j,
           att_em_i, att_em_j, gnn_bias, bn1_gamma, bn1_beta, bn2_gamma,
           bn2_beta, out_W, out_b):
    bsz = data.shape[0]
    f32 = jnp.float32
    data_pad = jnp.concatenate(
        [data, jnp.zeros((bsz, NODE, 1), f32)], axis=-1)
    ls3 = last_state.reshape(bsz, 1, NODE)
    ai = att_i.reshape(1, DIM)
    aj = att_j.reshape(1, DIM)
    aei = att_em_i.reshape(1, DIM)
    aej = att_em_j.reshape(1, DIM)
    w127 = lin_W[DIM - 1].reshape(1, DIM)
    mask8 = (jnp.arange(NODE)[None, :] // GRP
             != jnp.arange(NMASK)[:, None]).astype(f32)
    bias = gnn_bias.reshape(1, DIM)
    g1 = bn1_gamma.reshape(1, DIM)
    b1 = bn1_beta.reshape(1, DIM)
    g2 = bn2_gamma.reshape(1, DIM)
    b2 = bn2_beta.reshape(1, DIM)
    ow = out_W.reshape(1, DIM)
    ob = out_b.reshape(1, 1)

    sel = pl.pallas_call(
        _graph_kernel,
        out_shape=jax.ShapeDtypeStruct((NODE, NODE), f32),
    )(embedding)

    const2 = lambda shape: pl.BlockSpec(shape, lambda b: (0,) * len(shape))
    agg, s1, q1 = pl.pallas_call(
        _main_kernel,
        grid=(bsz,),
        in_specs=[
            pl.BlockSpec((1, NODE, DIM), lambda b: (b, 0, 0)),
            pl.BlockSpec((1, 1, NODE), lambda b: (b, 0, 0)),
            const2((NODE, NODE)),
            const2((NODE, DIM)),
            const2((DIM, DIM)),
            const2((1, DIM)),
            const2((1, DIM)),
            const2((1, DIM)),
            const2((1, DIM)),
            const2((1, DIM)),
            const2((NMASK, NODE)),
            const2((1, DIM)),
        ],
        out_specs=(
            pl.BlockSpec((1, NMASK, NODE, DIM), lambda b: (b, 0, 0, 0)),
            const2((1, DIM)),
            const2((1, DIM)),
        ),
        out_shape=(
            jax.ShapeDtypeStruct((bsz, NMASK, NODE, DIM), jnp.bfloat16),
            jax.ShapeDtypeStruct((1, DIM), f32),
            jax.ShapeDtypeStruct((1, DIM), f32),
        ),
    )(data_pad, ls3, sel, embedding, lin_W, ai, aj, aei, aej, w127, mask8,
      bias)

    out = pl.pallas_call(
        _bn_kernel,
        grid=(bsz,),
        in_specs=[
            pl.BlockSpec((1, NMASK, NODE, DIM), lambda b: (b, 0, 0, 0)),
            const2((1, DIM)),
            const2((1, DIM)),
            const2((NODE, DIM)),
            const2((1, DIM)),
            const2((1, DIM)),
            const2((1, DIM)),
            const2((1, DIM)),
            const2((1, DIM)),
            const2((1, 1)),
        ],
        out_specs=pl.BlockSpec((bsz, NODE), lambda b: (0, 0)),
        out_shape=jax.ShapeDtypeStruct((bsz, NODE), f32),
        scratch_shapes=[
            pltpu.VMEM((BSZ, NODE, DIM), f32),
            pltpu.VMEM((1, DIM), f32),
            pltpu.VMEM((1, DIM), f32),
        ],
    )(agg, s1, q1, embedding, g1, b1, g2, b2, ow, ob)
    return out


# speedup vs baseline: 1.0064x; 1.0064x over previous
import jax
import jax.numpy as jnp
from jax.experimental import pallas as pl
from jax.experimental.pallas import tpu as pltpu

NODE = 128
NMASK = 8
GRP = NODE // NMASK
DIM = 128
TOPK = 20
BSZ = 64
HIGH = jax.lax.Precision.HIGHEST


def _b16(v):
    return v.astype(jnp.bfloat16).astype(jnp.float32)


def _main_kernel(data_ref, ls_ref, emb_ref, linw_ref, ai_ref, aj_ref,
                 aei_ref, aej_ref, w127_ref, mask8_ref, bias_ref,
                 agg_ref, s1_ref, q1_ref, s_sc):
    b = pl.program_id(0)

    @pl.when(b == 0)
    def _graph():
        w = emb_ref[...]
        g = jax.lax.dot_general(w, w, (((1,), (1,)), ((), ())))
        nrm = jnp.sqrt(jnp.sum(w * w, axis=1))
        cos = g / (nrm[:, None] * nrm[None, :])
        colid = jax.lax.broadcasted_iota(jnp.int32, (NODE, NODE), 1)
        sel0 = jnp.zeros((NODE, NODE), jnp.float32)
        work = cos
        for _ in range(TOPK):
            rowmax = jnp.max(work, axis=1)
            ismax = work == rowmax[:, None]
            jidx = jnp.min(jnp.where(ismax, colid, NODE), axis=1)
            pick = colid == jidx[:, None]
            sel0 = jnp.where(pick, 1.0, sel0)
            work = jnp.where(pick, -jnp.inf, work)
        s_sc[...] = sel0

    x = data_ref[0]
    base = jnp.dot(x, linw_ref[...], preferred_element_type=jnp.float32)
    ai = _b16(ai_ref[...])
    aj = _b16(aj_ref[...])
    wb = _b16(emb_ref[...])
    ei = jnp.sum(wb * _b16(aei_ref[...]), axis=1)
    ej = jnp.sum(wb * _b16(aej_ref[...]), axis=1)
    w127 = w127_ref[...]
    ls = ls_ref[0, 0]
    sel = s_sc[...]
    mask8 = mask8_ref[...]
    bias = bias_ref[...]
    ssum = jnp.zeros((NODE,), jnp.float32)
    sq = jnp.zeros((NODE,), jnp.float32)
    for m in range(NMASK):
        c = ls * mask8[m]
        xl = base + c[:, None] * w127
        xb = _b16(xl)
        si = jnp.sum(xb * ai, axis=1) + ei
        sj = jnp.sum(xb * aj, axis=1) + ej
        a = si[:, None] + sj[None, :]
        a = jnp.where(a >= 0, a, 0.2 * a)
        a = jnp.where(sel > 0, a, -1e30)
        amax = jnp.max(a, axis=1)
        ex = jnp.exp(a - amax[:, None])
        den = jnp.sum(ex, axis=1)
        rcp = 1.0 / (den + 1e-16)
        agg = jnp.dot(ex, xl, preferred_element_type=jnp.float32,
                      precision=HIGH) * rcp[:, None] + bias
        agg_ref[0, m] = agg
        ssum = ssum + jnp.sum(agg, axis=0)
        sq = sq + jnp.sum(agg * agg, axis=0)

    @pl.when(b == 0)
    def _init():
        s1_ref[...] = jnp.zeros_like(s1_ref)
        q1_ref[...] = jnp.zeros_like(q1_ref)

    s1_ref[...] += ssum[None, :]
    q1_ref[...] += sq[None, :]


def _bn_kernel(agg_ref, s1_ref, q1_ref, emb_ref, g1_ref, b1_ref, g2_ref,
               b2_ref, ow_ref, ob_ref, out_ref, usel_sc, s2_sc, q2_sc):
    b = pl.program_id(0)
    n = float(BSZ * NMASK * NODE)
    mean = s1_ref[...] / n
    var = q1_ref[...] / n - mean * mean
    inv = g1_ref[...] / jnp.sqrt(var + 1e-5)
    sh = b1_ref[...] - mean * inv
    emb = emb_ref[...]
    ssum = jnp.zeros((1, DIM), jnp.float32)
    sq = jnp.zeros((1, DIM), jnp.float32)
    for m in range(NMASK):
        h = jnp.maximum(agg_ref[0, m] * inv + sh, 0.0)
        u = h * emb
        ssum = ssum + jnp.sum(u, axis=0)[None, :]
        sq = sq + jnp.sum(u * u, axis=0)[None, :]
        usel_sc[b, m * GRP:(m + 1) * GRP, :] = u[m * GRP:(m + 1) * GRP, :]

    @pl.when(b == 0)
    def _init():
        s2_sc[...] = jnp.zeros_like(s2_sc)
        q2_sc[...] = jnp.zeros_like(q2_sc)

    s2_sc[...] += ssum
    q2_sc[...] += sq

    @pl.when(b == BSZ - 1)
    def _final():
        mean2 = s2_sc[...] / n
        var2 = q2_sc[...] / n - mean2 * mean2
        inv2 = g2_ref[...] / jnp.sqrt(var2 + 1e-5)
        sh2 = b2_ref[...] - mean2 * inv2
        y = jnp.maximum(usel_sc[...] * inv2[None] + sh2[None], 0.0)
        yb = _b16(y)
        wb = _b16(ow_ref[...])
        out_ref[...] = jnp.sum(yb * wb[None], axis=2) + ob_ref[0, 0]


def kernel(data, org_edge_index, last_state, embedding, lin_W, att_i, att_j,
           att_em_i, att_em_j, gnn_bias, bn1_gamma, bn1_beta, bn2_gamma,
           bn2_beta, out_W, out_b):
    bsz = data.shape[0]
    f32 = jnp.float32
    data_pad = jnp.concatenate(
        [data, jnp.zeros((bsz, NODE, 1), f32)], axis=-1)
    ls3 = last_state.reshape(bsz, 1, NODE)
    ai = att_i.reshape(1, DIM)
    aj = att_j.reshape(1, DIM)
    aei = att_em_i.reshape(1, DIM)
    aej = att_em_j.reshape(1, DIM)
    w127 = lin_W[DIM - 1].reshape(1, DIM)
    mask8 = (jnp.arange(NODE)[None, :] // GRP
             != jnp.arange(NMASK)[:, None]).astype(f32)
    bias = gnn_bias.reshape(1, DIM)
    g1 = bn1_gamma.reshape(1, DIM)
    b1 = bn1_beta.reshape(1, DIM)
    g2 = bn2_gamma.reshape(1, DIM)
    b2 = bn2_beta.reshape(1, DIM)
    ow = out_W.reshape(1, DIM)
    ob = out_b.reshape(1, 1)

    const2 = lambda shape: pl.BlockSpec(shape, lambda b: (0,) * len(shape))
    agg, s1, q1 = pl.pallas_call(
        _main_kernel,
        grid=(bsz,),
        in_specs=[
            pl.BlockSpec((1, NODE, DIM), lambda b: (b, 0, 0)),
            pl.BlockSpec((1, 1, NODE), lambda b: (b, 0, 0)),
            const2((NODE, DIM)),
            const2((DIM, DIM)),
            const2((1, DIM)),
            const2((1, DIM)),
            const2((1, DIM)),
            const2((1, DIM)),
            const2((1, DIM)),
            const2((NMASK, NODE)),
            const2((1, DIM)),
        ],
        out_specs=(
            pl.BlockSpec((1, NMASK, NODE, DIM), lambda b: (b, 0, 0, 0)),
            const2((1, DIM)),
            const2((1, DIM)),
        ),
        out_shape=(
            jax.ShapeDtypeStruct((bsz, NMASK, NODE, DIM), f32),
            jax.ShapeDtypeStruct((1, DIM), f32),
            jax.ShapeDtypeStruct((1, DIM), f32),
        ),
        scratch_shapes=[pltpu.VMEM((NODE, NODE), f32)],
    )(data_pad, ls3, embedding, lin_W, ai, aj, aei, aej, w127, mask8, bias)

    out = pl.pallas_call(
        _bn_kernel,
        grid=(bsz,),
        in_specs=[
            pl.BlockSpec((1, NMASK, NODE, DIM), lambda b: (b, 0, 0, 0)),
            const2((1, DIM)),
            const2((1, DIM)),
            const2((NODE, DIM)),
            const2((1, DIM)),
            const2((1, DIM)),
            const2((1, DIM)),
            const2((1, DIM)),
            const2((1, DIM)),
            const2((1, 1)),
        ],
        out_specs=pl.BlockSpec((bsz, NODE), lambda b: (0, 0)),
        out_shape=jax.ShapeDtypeStruct((bsz, NODE), f32),
        scratch_shapes=[
            pltpu.VMEM((BSZ, NODE, DIM), f32),
            pltpu.VMEM((1, DIM), f32),
            pltpu.VMEM((1, DIM), f32),
        ],
    )(agg, s1, q1, embedding, g1, b1, g2, b2, ow, ob)
    return out


# speedup vs baseline: 1.0353x; 1.0287x over previous
import jax
import jax.numpy as jnp
from jax.experimental import pallas as pl
from jax.experimental.pallas import tpu as pltpu

NODE = 128
NMASK = 8
GRP = NODE // NMASK
DIM = 128
TOPK = 20
BSZ = 64
HIGH = jax.lax.Precision.HIGHEST


def _b16(v):
    return v.astype(jnp.bfloat16).astype(jnp.float32)


def _fused_kernel(data_ref, ls_ref, emb_ref, linw_ref, ai_ref, aj_ref,
                  aei_ref, aej_ref, w127_ref, mask8_ref, bias_ref,
                  g1_ref, b1_ref, g2_ref, b2_ref, ow_ref, ob_ref,
                  out_ref, s_sc, agg_sc, s1_sc, q1_sc, usel_sc, s2_sc, q2_sc):
    step = pl.program_id(0)

    @pl.when(step == 0)
    def _graph():
        w = emb_ref[...]
        g = jax.lax.dot_general(w, w, (((1,), (1,)), ((), ())))
        nrm = jnp.sqrt(jnp.sum(w * w, axis=1))
        cos = g / (nrm[:, None] * nrm[None, :])
        colid = jax.lax.broadcasted_iota(jnp.int32, (NODE, NODE), 1)
        sel0 = jnp.zeros((NODE, NODE), jnp.float32)
        work = cos
        for _ in range(TOPK):
            rowmax = jnp.max(work, axis=1)
            ismax = work == rowmax[:, None]
            jidx = jnp.min(jnp.where(ismax, colid, NODE), axis=1)
            pick = colid == jidx[:, None]
            sel0 = jnp.where(pick, 1.0, sel0)
            work = jnp.where(pick, -jnp.inf, work)
        s_sc[...] = sel0
        s1_sc[...] = jnp.zeros_like(s1_sc)
        q1_sc[...] = jnp.zeros_like(q1_sc)
        s2_sc[...] = jnp.zeros_like(s2_sc)
        q2_sc[...] = jnp.zeros_like(q2_sc)

    @pl.when(step < BSZ)
    def _phase1():
        b = step
        x = data_ref[0]
        base = jnp.dot(x, linw_ref[...], preferred_element_type=jnp.float32)
        ai = _b16(ai_ref[...])
        aj = _b16(aj_ref[...])
        wb = _b16(emb_ref[...])
        ei = jnp.sum(wb * _b16(aei_ref[...]), axis=1)
        ej = jnp.sum(wb * _b16(aej_ref[...]), axis=1)
        w127 = w127_ref[...]
        ls = ls_ref[0, 0]
        sel = s_sc[...]
        mask8 = mask8_ref[...]
        bias = bias_ref[...]
        ssum = jnp.zeros((NODE,), jnp.float32)
        sq = jnp.zeros((NODE,), jnp.float32)
        for m in range(NMASK):
            c = ls * mask8[m]
            xl = base + c[:, None] * w127
            xb = _b16(xl)
            si = jnp.sum(xb * ai, axis=1) + ei
            sj = jnp.sum(xb * aj, axis=1) + ej
            a = si[:, None] + sj[None, :]
            a = jnp.where(a >= 0, a, 0.2 * a)
            a = jnp.where(sel > 0, a, -1e30)
            amax = jnp.max(a, axis=1)
            ex = jnp.exp(a - amax[:, None])
            den = jnp.sum(ex, axis=1)
            rcp = 1.0 / (den + 1e-16)
            agg = jnp.dot(ex, xl, preferred_element_type=jnp.float32,
                          precision=HIGH) * rcp[:, None] + bias
            agg_sc[b, m] = agg
            ssum = ssum + jnp.sum(agg, axis=0)
            sq = sq + jnp.sum(agg * agg, axis=0)
        s1_sc[...] += ssum[None, :]
        q1_sc[...] += sq[None, :]

    @pl.when(step >= BSZ)
    def _phase2():
        b = step - BSZ
        n = float(BSZ * NMASK * NODE)
        mean = s1_sc[...] / n
        var = q1_sc[...] / n - mean * mean
        inv = g1_ref[...] / jnp.sqrt(var + 1e-5)
        sh = b1_ref[...] - mean * inv
        emb = emb_ref[...]
        ssum = jnp.zeros((1, DIM), jnp.float32)
        sq = jnp.zeros((1, DIM), jnp.float32)
        for m in range(NMASK):
            h = jnp.maximum(agg_sc[b, m] * inv + sh, 0.0)
            u = h * emb
            ssum = ssum + jnp.sum(u, axis=0)[None, :]
            sq = sq + jnp.sum(u * u, axis=0)[None, :]
            usel_sc[b, m * GRP:(m + 1) * GRP, :] = u[m * GRP:(m + 1) * GRP, :]
        s2_sc[...] += ssum
        q2_sc[...] += sq

        @pl.when(b == BSZ - 1)
        def _final():
            mean2 = s2_sc[...] / n
            var2 = q2_sc[...] / n - mean2 * mean2
            inv2 = g2_ref[...] / jnp.sqrt(var2 + 1e-5)
            sh2 = b2_ref[...] - mean2 * inv2
            y = jnp.maximum(usel_sc[...] * inv2[None] + sh2[None], 0.0)
            yb = _b16(y)
            wb = _b16(ow_ref[...])
            out_ref[...] = jnp.sum(yb * wb[None], axis=2) + ob_ref[0, 0]


def kernel(data, org_edge_index, last_state, embedding, lin_W, att_i, att_j,
           att_em_i, att_em_j, gnn_bias, bn1_gamma, bn1_beta, bn2_gamma,
           bn2_beta, out_W, out_b):
    bsz = data.shape[0]
    f32 = jnp.float32
    data_pad = jnp.concatenate(
        [data, jnp.zeros((bsz, NODE, 1), f32)], axis=-1)
    ls3 = last_state.reshape(bsz, 1, NODE)
    ai = att_i.reshape(1, DIM)
    aj = att_j.reshape(1, DIM)
    aei = att_em_i.reshape(1, DIM)
    aej = att_em_j.reshape(1, DIM)
    w127 = lin_W[DIM - 1].reshape(1, DIM)
    mask8 = (jnp.arange(NODE)[None, :] // GRP
             != jnp.arange(NMASK)[:, None]).astype(f32)
    bias = gnn_bias.reshape(1, DIM)
    g1 = bn1_gamma.reshape(1, DIM)
    b1 = bn1_beta.reshape(1, DIM)
    g2 = bn2_gamma.reshape(1, DIM)
    b2 = bn2_beta.reshape(1, DIM)
    ow = out_W.reshape(1, DIM)
    ob = out_b.reshape(1, 1)

    const2 = lambda shape: pl.BlockSpec(shape, lambda s: (0,) * len(shape))
    out = pl.pallas_call(
        _fused_kernel,
        grid=(2 * bsz,),
        in_specs=[
            pl.BlockSpec((1, NODE, DIM), lambda s: (s % BSZ, 0, 0)),
            pl.BlockSpec((1, 1, NODE), lambda s: (s % BSZ, 0, 0)),
            const2((NODE, DIM)),
            const2((DIM, DIM)),
            const2((1, DIM)),
            const2((1, DIM)),
            const2((1, DIM)),
            const2((1, DIM)),
            const2((1, DIM)),
            const2((NMASK, NODE)),
            const2((1, DIM)),
            const2((1, DIM)),
            const2((1, DIM)),
            const2((1, DIM)),
            const2((1, DIM)),
            const2((1, DIM)),
            const2((1, 1)),
        ],
        out_specs=pl.BlockSpec((bsz, NODE), lambda s: (0, 0)),
        out_shape=jax.ShapeDtypeStruct((bsz, NODE), f32),
        scratch_shapes=[
            pltpu.VMEM((NODE, NODE), f32),
            pltpu.VMEM((BSZ, NMASK, NODE, DIM), f32),
            pltpu.VMEM((1, DIM), f32),
            pltpu.VMEM((1, DIM), f32),
            pltpu.VMEM((BSZ, NODE, DIM), f32),
            pltpu.VMEM((1, DIM), f32),
            pltpu.VMEM((1, DIM), f32),
        ],
    )(data_pad, ls3, embedding, lin_W, ai, aj, aei, aej, w127, mask8, bias,
      g1, b1, g2, b2, ow, ob)
    return out
